# baseline (speedup 1.0000x reference)
import functools

import jax
import jax.numpy as jnp
from jax import lax
from jax.experimental import pallas as pl
from jax.experimental.pallas import tpu as pltpu
from jax.experimental.pallas import tpu_sc as plsc

N = 8388608
NW = 32
PER_W = N // NW
CH = 2048
NCHUNK = PER_W // CH
GROUPS = CH // 128
NSLOT = 8
assert NCHUNK % NSLOT == 0
assert NSLOT * (CH + 4 * CH) <= 131071

_mesh = plsc.VectorSubcoreMesh(core_axis_name="c", subcore_axis_name="s")


def _sc_body(x_hbm, out_hbm, xbuf, obuf, insem, outsem):
    wid = lax.axis_index("s") * 2 + lax.axis_index("c")
    xbase = wid * PER_W
    obase = wid * (PER_W * 4)

    def compute_chunk(slot):
        @plsc.parallel_loop(0, GROUPS, step=1, unroll=2)
        def group_body(g):
            for u in range(8):
                v = xbuf[pl.ds(slot * CH + g * 128 + u * 16, 16)]
                a = jnp.abs(v)
                one = jnp.int32(1)
                zero = jnp.int32(0)
                s1 = jnp.where(a > 100.0, one, zero)
                s2 = jnp.where(a > 500.0, one, zero)
                s3 = jnp.where(a > 1000.0, one, zero)
                base = slot * CH * 4 + g * 512 + u * 16
                obuf[pl.ds(base, 16)] = one - s1
                obuf[pl.ds(base + 128, 16)] = s1 - s2
                obuf[pl.ds(base + 256, 16)] = s2 - s3
                obuf[pl.ds(base + 384, 16)] = s3

    def in_copy(g, slot):
        return pltpu.make_async_copy(
            x_hbm.at[pl.ds(xbase + g * CH, CH)],
            xbuf.at[pl.ds(slot * CH, CH)], insem.at[slot])

    def out_copy(g, slot):
        return pltpu.make_async_copy(
            obuf.at[pl.ds(slot * CH * 4, CH * 4)],
            out_hbm.at[pl.ds(obase + g * CH * 4, CH * 4)],
            outsem.at[slot])

    for slot in range(NSLOT):
        in_copy(slot, slot).start()

    def step(gg, _):
        for slot in range(NSLOT):
            g = NSLOT * gg + slot
            in_copy(g, slot).wait()

            @pl.when(gg > 0)
            def _wait_prev_out():
                out_copy(g - NSLOT, slot).wait()

            compute_chunk(slot)
            out_copy(g, slot).start()

            @pl.when(g + NSLOT < NCHUNK)
            def _prefetch_in():
                in_copy(g + NSLOT, slot).start()
        return 0

    lax.fori_loop(0, NCHUNK // NSLOT, step, 0)
    for slot in range(NSLOT):
        out_copy(NCHUNK - NSLOT + slot, slot).wait()


@jax.jit
def kernel(x):
    sc_call = functools.partial(
        pl.kernel,
        mesh=_mesh,
        out_type=jax.ShapeDtypeStruct((N * 4,), jnp.int32),
        scratch_types=[
            pltpu.VMEM((NSLOT * CH,), jnp.float32),
            pltpu.VMEM((NSLOT * CH * 4,), jnp.int32),
            pltpu.SemaphoreType.DMA((NSLOT,)),
            pltpu.SemaphoreType.DMA((NSLOT,)),
        ],
    )(_sc_body)
    out = sc_call(x)
    return (out.reshape(N // 128, 4, 128)
               .transpose(0, 2, 1)
               .reshape(N, 4))


# speedup vs baseline: 1.0651x; 1.0651x over previous
import functools

import jax
import jax.numpy as jnp
from jax import lax
from jax.experimental import pallas as pl
from jax.experimental.pallas import tpu as pltpu
from jax.experimental.pallas import tpu_sc as plsc

N = 8388608
NW = 32
PER_W = N // NW
CH = 4096
NCHUNK = PER_W // CH
GROUPS = CH // 128
NSLOT = 4
assert NCHUNK % NSLOT == 0
assert NSLOT * (CH + 4 * CH) <= 131071

_mesh = plsc.VectorSubcoreMesh(core_axis_name="c", subcore_axis_name="s")


def _sc_body(x_hbm, out_hbm, xbuf, obuf, insem, outsem):
    wid = lax.axis_index("s") * 2 + lax.axis_index("c")
    xbase = wid * PER_W
    obase = wid * (PER_W * 4)

    def compute_chunk(slot):
        @plsc.parallel_loop(0, GROUPS, step=1, unroll=1)
        def group_body(g):
            for u in range(8):
                v = xbuf[pl.ds(slot * CH + g * 128 + u * 16, 16)]
                a = jnp.abs(v)
                one = jnp.int32(1)
                zero = jnp.int32(0)
                s1 = jnp.where(a > 100.0, one, zero)
                s2 = jnp.where(a > 500.0, one, zero)
                s3 = jnp.where(a > 1000.0, one, zero)
                base = slot * CH * 4 + g * 512 + u * 16
                obuf[pl.ds(base, 16)] = one - s1
                obuf[pl.ds(base + 128, 16)] = s1 - s2
                obuf[pl.ds(base + 256, 16)] = s2 - s3
                obuf[pl.ds(base + 384, 16)] = s3

    def in_copy(g, slot):
        return pltpu.make_async_copy(
            x_hbm.at[pl.ds(xbase + g * CH, CH)],
            xbuf.at[pl.ds(slot * CH, CH)], insem.at[slot])

    def out_copy(g, slot):
        return pltpu.make_async_copy(
            obuf.at[pl.ds(slot * CH * 4, CH * 4)],
            out_hbm.at[pl.ds(obase + g * CH * 4, CH * 4)],
            outsem.at[slot])

    for slot in range(NSLOT):
        in_copy(slot, slot).start()

    def step(gg, _):
        for slot in range(NSLOT):
            g = NSLOT * gg + slot
            in_copy(g, slot).wait()

            @pl.when(gg > 0)
            def _wait_prev_out():
                out_copy(g - NSLOT, slot).wait()

            compute_chunk(slot)
            out_copy(g, slot).start()

            @pl.when(g + NSLOT < NCHUNK)
            def _prefetch_in():
                in_copy(g + NSLOT, slot).start()
        return 0

    lax.fori_loop(0, NCHUNK // NSLOT, step, 0)
    for slot in range(NSLOT):
        out_copy(NCHUNK - NSLOT + slot, slot).wait()


@jax.jit
def kernel(x):
    sc_call = functools.partial(
        pl.kernel,
        mesh=_mesh,
        out_type=jax.ShapeDtypeStruct((N * 4,), jnp.int32),
        scratch_types=[
            pltpu.VMEM((NSLOT * CH,), jnp.float32),
            pltpu.VMEM((NSLOT * CH * 4,), jnp.int32),
            pltpu.SemaphoreType.DMA((NSLOT,)),
            pltpu.SemaphoreType.DMA((NSLOT,)),
        ],
    )(_sc_body)
    out = sc_call(x)
    return (out.reshape(N // 128, 4, 128)
               .transpose(0, 2, 1)
               .reshape(N, 4))
